# baseline (speedup 1.0000x reference)
import functools

import jax
import jax.numpy as jnp
from jax import lax
from jax.experimental import pallas as pl
from jax.experimental.pallas import tpu as pltpu
from jax.experimental.pallas import tpu_sc as plsc

N = 50000
KC = 4
XS = 128
HS = 128
E = 800000

NP = 50176
DUMMY = N
NE_PAD = 819200
EROWS = 6400
NCORE = 2
NSUB = 16
EPW = NE_PAD // (NCORE * NSUB)
NP_SLICE = NP // NSUB

@functools.lru_cache(maxsize=None)
def _get_sc_degrees():
    mesh = plsc.VectorSubcoreMesh(
        core_axis_name="c", subcore_axis_name="s", num_cores=NCORE,
        num_subcores=NSUB)
    return functools.partial(
        pl.kernel,
        out_type=jax.ShapeDtypeStruct((NCORE * 2 * NP,), jnp.float32),
        mesh=mesh,
        scratch_types=[
            pltpu.VMEM((EPW,), jnp.int32),
            pltpu.VMEM((EPW,), jnp.int32),
            pltpu.VMEM((EPW,), jnp.float32),
            pltpu.VMEM((NP_SLICE,), jnp.float32),
            pltpu.VMEM_SHARED((NP,), jnp.float32),
            pltpu.VMEM_SHARED((NP,), jnp.float32),
        ],
    )(_sc_degrees_body)


def _sc_degrees_body(src_hbm, dst_hbm, zeros_hbm, ones_hbm, deg_out,
                     idx_src, idx_dst, ones_v, bounce, acc_o, acc_i):
    c = lax.axis_index("c")
    s = lax.axis_index("s")
    off = s * NP_SLICE

    pltpu.sync_copy(zeros_hbm.at[pl.ds(off, NP_SLICE)], bounce)
    pltpu.sync_copy(bounce, acc_o.at[pl.ds(off, NP_SLICE)])
    pltpu.sync_copy(bounce, acc_i.at[pl.ds(off, NP_SLICE)])

    base = (c * NSUB + s) * EPW
    pltpu.sync_copy(src_hbm.at[pl.ds(base, EPW)], idx_src)
    pltpu.sync_copy(dst_hbm.at[pl.ds(base, EPW)], idx_dst)
    pltpu.sync_copy(ones_hbm, ones_v)
    plsc.subcore_barrier()

    pltpu.sync_copy(ones_v, acc_o.at[idx_src], add=True)
    pltpu.sync_copy(ones_v, acc_i.at[idx_dst], add=True)
    plsc.subcore_barrier()
    pltpu.sync_copy(acc_o.at[pl.ds(off, NP_SLICE)], bounce)
    pltpu.sync_copy(bounce, deg_out.at[pl.ds((c * 2) * NP + off, NP_SLICE)])
    pltpu.sync_copy(acc_i.at[pl.ds(off, NP_SLICE)], bounce)
    pltpu.sync_copy(bounce, deg_out.at[pl.ds((c * 2 + 1) * NP + off, NP_SLICE)])


@functools.lru_cache(maxsize=None)
def _get_sc_scatter():
    mesh = plsc.VectorSubcoreMesh(
        core_axis_name="c", subcore_axis_name="s", num_cores=NCORE,
        num_subcores=NSUB)
    return functools.partial(
        pl.kernel,
        out_type=jax.ShapeDtypeStruct((NCORE * NP,), jnp.float32),
        mesh=mesh,
        scratch_types=[
            pltpu.VMEM((EPW,), jnp.int32),
            pltpu.VMEM((EPW,), jnp.int32),
            pltpu.VMEM((EPW,), jnp.float32),
            pltpu.VMEM((NP_SLICE,), jnp.float32),
            pltpu.VMEM_SHARED((NP,), jnp.float32),
            pltpu.SemaphoreType.DMA,
        ],
    )(_sc_scatter_body)


def _sc_scatter_body(src_hbm, dst_hbm, s_hbm, zeros_hbm, agg_out,
                     idx_src, idx_dst, vals, bounce, acc, sem):
    c = lax.axis_index("c")
    s = lax.axis_index("s")
    off = s * NP_SLICE

    pltpu.sync_copy(zeros_hbm.at[pl.ds(off, NP_SLICE)], bounce)
    pltpu.sync_copy(bounce, acc.at[pl.ds(off, NP_SLICE)])

    base = (c * NSUB + s) * EPW
    pltpu.sync_copy(src_hbm.at[pl.ds(base, EPW)], idx_src)
    pltpu.sync_copy(dst_hbm.at[pl.ds(base, EPW)], idx_dst)
    plsc.subcore_barrier()

    pltpu.async_copy(s_hbm.at[idx_src], vals, sem).wait()
    pltpu.sync_copy(vals, acc.at[idx_dst], add=True)
    plsc.subcore_barrier()
    pltpu.sync_copy(acc.at[pl.ds(off, NP_SLICE)], bounce)
    pltpu.sync_copy(bounce, agg_out.at[pl.ds(c * NP + off, NP_SLICE)])


TB = 1000


def _treelstm_body(x_ref, c_ref, wiou_ref, b_ref, uf_ref, ufb_ref,
                   uiou_ref, w1_ref, oda_ref, odb_ref, s_ref,
                   h_acc, c_acc):
    k = pl.program_id(1)
    bf16 = jnp.bfloat16
    x = x_ref[...]
    cc = c_ref[...]
    b = b_ref[...]
    iou = jnp.dot(x.astype(bf16), wiou_ref[...].astype(bf16),
                  preferred_element_type=jnp.float32) + b
    i_ = jax.nn.sigmoid(iou[:, :HS])
    o_ = jax.nn.sigmoid(iou[:, HS:2 * HS])
    u_ = jnp.tanh(iou[:, 2 * HS:])
    c_leaf = i_ * u_ + cc
    h_leaf = o_ * jnp.tanh(c_leaf)
    f = jax.nn.sigmoid(
        jnp.dot(h_leaf.astype(bf16), uf_ref[...].astype(bf16),
                preferred_element_type=jnp.float32)
        + ufb_ref[...])
    fc = f * c_leaf

    @pl.when(k == 0)
    def _():
        h_acc[...] = h_leaf
        c_acc[...] = fc

    @pl.when(k > 0)
    def _():
        h_acc[...] += h_leaf
        c_acc[...] += fc

    @pl.when(k == KC - 1)
    def _():
        iou_r = jnp.dot(h_acc[...].astype(bf16), uiou_ref[...].astype(bf16),
                        preferred_element_type=jnp.float32) + b
        ir = jax.nn.sigmoid(iou_r[:, :HS])
        orr = jax.nn.sigmoid(iou_r[:, HS:2 * HS])
        ur = jnp.tanh(iou_r[:, 2 * HS:])
        c_root = ir * ur + c_acc[...]
        feat = orr * jnp.tanh(c_root)
        g = jnp.dot(feat, w1_ref[...], preferred_element_type=jnp.float32)
        od = oda_ref[...] + odb_ref[...]
        s_ref[...] = g * lax.rsqrt(jnp.maximum(od, 1.0))


def _treelstm(x2d, c2d, wiou_t, b2d, uf_t, ufb2d, uiou_t, w1, oda, odb):
    full = lambda shp: pl.BlockSpec(shp, lambda i, k: (0, 0))
    return pl.pallas_call(
        _treelstm_body,
        grid=(N // TB, KC),
        in_specs=[
            pl.BlockSpec((TB, XS), lambda i, k: (i, k + 1)),
            pl.BlockSpec((TB, HS), lambda i, k: (i, k + 1)),
            full((XS, 3 * HS)),
            full((1, 3 * HS)),
            full((HS, HS)),
            full((1, HS)),
            full((HS, 3 * HS)),
            full((HS, 1)),
            pl.BlockSpec((TB, 1), lambda i, k: (i, 0)),
            pl.BlockSpec((TB, 1), lambda i, k: (i, 0)),
        ],
        out_specs=pl.BlockSpec((TB, 1), lambda i, k: (i, 0)),
        out_shape=jax.ShapeDtypeStruct((N, 1), jnp.float32),
        scratch_shapes=[
            pltpu.VMEM((TB, HS), jnp.float32),
            pltpu.VMEM((TB, HS), jnp.float32),
        ],
    )(x2d, c2d, wiou_t, b2d, uf_t, ufb2d, uiou_t, w1, oda, odb)


FB = NP // 8


def _final_body(p0_ref, p1_ref, ida_ref, idb_ref, bc_ref, out_ref):
    agg = p0_ref[...] + p1_ref[...]
    ind = jnp.maximum(ida_ref[...] + idb_ref[...], 1.0)
    out_ref[...] = jax.nn.sigmoid(agg * lax.rsqrt(ind) + bc_ref[0, 0])


def _final(p0, p1, ida, idb, bc):
    bs = pl.BlockSpec((FB, 1), lambda i: (i, 0))
    return pl.pallas_call(
        _final_body,
        grid=(NP // FB,),
        in_specs=[bs, bs, bs, bs, pl.BlockSpec((1, 1), lambda i: (0, 0))],
        out_specs=bs,
        out_shape=jax.ShapeDtypeStruct((NP, 1), jnp.float32),
    )(p0, p1, ida, idb, bc)


def kernel(ast_x, h, c, cfg_edge_index, W_iou, U_iou, b_iou, U_f_w, U_f_b,
           gcn_w, gcn_b, lin1_w, lin1_b, lin2_w, lin2_b):
    del h
    f32 = jnp.float32

    m32 = lin1_w.T @ lin2_w.T
    w1 = gcn_w @ m32
    bconst = (gcn_b @ m32 + lin1_b @ lin2_w.T + lin2_b).reshape(1, 1)

    pad = jnp.full((2, NE_PAD - E), DUMMY, cfg_edge_index.dtype)
    edges = jnp.concatenate([cfg_edge_index, pad], axis=1).astype(jnp.int32)
    src_flat = edges[0]
    dst_flat = edges[1]

    zeros_np = jnp.zeros((NP,), f32)
    ones_pay = jnp.ones((EPW,), f32)

    deg = _get_sc_degrees()(src_flat, dst_flat, zeros_np,
                            ones_pay).reshape(NCORE, 2, NP)

    x2d = ast_x.reshape(N, (KC + 1) * XS)
    c2d = c.reshape(N, (KC + 1) * HS)
    oda = deg[0, 0, :N].reshape(N, 1)
    odb = deg[1, 0, :N].reshape(N, 1)
    s50 = _treelstm(x2d, c2d,
                    W_iou.T, b_iou.reshape(1, 3 * HS),
                    U_f_w.T, U_f_b.reshape(1, HS),
                    U_iou.T, w1, oda, odb)

    s_pad = jnp.concatenate([s50[:, 0], jnp.zeros((NP - N,), f32)])

    agg = _get_sc_scatter()(src_flat, dst_flat, s_pad,
                            zeros_np).reshape(NCORE, NP)

    out = _final(agg[0].reshape(NP, 1), agg[1].reshape(NP, 1),
                 deg[0, 1].reshape(NP, 1), deg[1, 1].reshape(NP, 1),
                 bconst.astype(f32))
    return out[:N]


# speedup vs baseline: 1.2075x; 1.2075x over previous
import functools

import jax
import jax.numpy as jnp
from jax import lax
from jax.experimental import pallas as pl
from jax.experimental.pallas import tpu as pltpu
from jax.experimental.pallas import tpu_sc as plsc

N = 50000
KC = 4
XS = 128
HS = 128
E = 800000

NP = 50176
DUMMY = N
NE_PAD = 819200
EROWS = 6400
NCORE = 2
NSUB = 16
EPW = NE_PAD // (NCORE * NSUB)
NP_SLICE = NP // NSUB

@functools.lru_cache(maxsize=None)
def _get_sc_degrees():
    mesh = plsc.VectorSubcoreMesh(
        core_axis_name="c", subcore_axis_name="s", num_cores=NCORE,
        num_subcores=NSUB)
    return functools.partial(
        pl.kernel,
        out_type=jax.ShapeDtypeStruct((NCORE * 2 * NP,), jnp.float32),
        mesh=mesh,
        scratch_types=[
            pltpu.VMEM((EPW,), jnp.int32),
            pltpu.VMEM((EPW,), jnp.int32),
            pltpu.VMEM((EPW,), jnp.float32),
            pltpu.VMEM((NP_SLICE,), jnp.float32),
            pltpu.VMEM_SHARED((NP,), jnp.float32),
            pltpu.VMEM_SHARED((NP,), jnp.float32),
        ],
    )(_sc_degrees_body)


def _sc_degrees_body(src_hbm, dst_hbm, zeros_hbm, ones_hbm, deg_out,
                     idx_src, idx_dst, ones_v, bounce, acc_o, acc_i):
    c = lax.axis_index("c")
    s = lax.axis_index("s")
    off = s * NP_SLICE

    pltpu.sync_copy(zeros_hbm.at[pl.ds(off, NP_SLICE)], bounce)
    pltpu.sync_copy(bounce, acc_o.at[pl.ds(off, NP_SLICE)])
    pltpu.sync_copy(bounce, acc_i.at[pl.ds(off, NP_SLICE)])

    base = (c * NSUB + s) * EPW
    pltpu.sync_copy(src_hbm.at[pl.ds(base, EPW)], idx_src)
    pltpu.sync_copy(dst_hbm.at[pl.ds(base, EPW)], idx_dst)
    pltpu.sync_copy(ones_hbm, ones_v)
    plsc.subcore_barrier()

    pltpu.sync_copy(ones_v, acc_o.at[idx_src], add=True)
    pltpu.sync_copy(ones_v, acc_i.at[idx_dst], add=True)
    plsc.subcore_barrier()
    pltpu.sync_copy(acc_o.at[pl.ds(off, NP_SLICE)], bounce)
    pltpu.sync_copy(bounce, deg_out.at[pl.ds((c * 2) * NP + off, NP_SLICE)])
    pltpu.sync_copy(acc_i.at[pl.ds(off, NP_SLICE)], bounce)
    pltpu.sync_copy(bounce, deg_out.at[pl.ds((c * 2 + 1) * NP + off, NP_SLICE)])


@functools.lru_cache(maxsize=None)
def _get_sc_scatter():
    mesh = plsc.VectorSubcoreMesh(
        core_axis_name="c", subcore_axis_name="s", num_cores=NCORE,
        num_subcores=NSUB)
    return functools.partial(
        pl.kernel,
        out_type=jax.ShapeDtypeStruct((NCORE * NP,), jnp.float32),
        mesh=mesh,
        scratch_types=[
            pltpu.VMEM((EPW,), jnp.int32),
            pltpu.VMEM((EPW,), jnp.int32),
            pltpu.VMEM((EPW,), jnp.float32),
            pltpu.VMEM((NP_SLICE,), jnp.float32),
            pltpu.VMEM_SHARED((NP,), jnp.float32),
            pltpu.SemaphoreType.DMA,
        ],
    )(_sc_scatter_body)


def _sc_scatter_body(src_hbm, dst_hbm, s_hbm, zeros_hbm, agg_out,
                     idx_src, idx_dst, vals, bounce, acc, sem):
    c = lax.axis_index("c")
    s = lax.axis_index("s")
    off = s * NP_SLICE

    pltpu.sync_copy(zeros_hbm.at[pl.ds(off, NP_SLICE)], bounce)
    pltpu.sync_copy(bounce, acc.at[pl.ds(off, NP_SLICE)])

    base = (c * NSUB + s) * EPW
    pltpu.sync_copy(src_hbm.at[pl.ds(base, EPW)], idx_src)
    pltpu.sync_copy(dst_hbm.at[pl.ds(base, EPW)], idx_dst)
    plsc.subcore_barrier()

    pltpu.async_copy(s_hbm.at[idx_src], vals, sem).wait()
    pltpu.sync_copy(vals, acc.at[idx_dst], add=True)
    plsc.subcore_barrier()
    pltpu.sync_copy(acc.at[pl.ds(off, NP_SLICE)], bounce)
    pltpu.sync_copy(bounce, agg_out.at[pl.ds(c * NP + off, NP_SLICE)])


TB = 2000


def _sigmoid(x):
    return jnp.tanh(x * 0.5) * 0.5 + 0.5


def _treelstm_body(x_ref, c_ref, wiou_ref, b_ref, uf_ref, ufb_ref,
                   uiou_ref, w1_ref, oda_ref, odb_ref, s_ref):
    bf16 = jnp.bfloat16
    b = b_ref[...]
    wiou = wiou_ref[...].astype(bf16)
    uf = uf_ref[...].astype(bf16)
    h_tild = None
    c_red = None
    for k in range(KC):
        lo = (k + 1) * HS
        x = x_ref[:, lo:lo + HS].astype(bf16)
        cc = c_ref[:, lo:lo + HS].astype(bf16)
        iou = (jnp.dot(x, wiou,
                       preferred_element_type=jnp.float32) + b).astype(bf16)
        i_ = _sigmoid(iou[:, :HS])
        o_ = _sigmoid(iou[:, HS:2 * HS])
        u_ = jnp.tanh(iou[:, 2 * HS:])
        c_leaf = i_ * u_ + cc
        h_leaf = o_ * jnp.tanh(c_leaf)
        f = _sigmoid(
            (jnp.dot(h_leaf, uf, preferred_element_type=jnp.float32)
             + ufb_ref[...]).astype(bf16))
        fc = f * c_leaf
        h_tild = h_leaf if h_tild is None else h_tild + h_leaf
        c_red = fc if c_red is None else c_red + fc

    iou_r = (jnp.dot(h_tild, uiou_ref[...].astype(bf16),
                     preferred_element_type=jnp.float32) + b).astype(bf16)
    ir = _sigmoid(iou_r[:, :HS])
    orr = _sigmoid(iou_r[:, HS:2 * HS])
    ur = jnp.tanh(iou_r[:, 2 * HS:])
    c_root = ir * ur + c_red
    feat = orr * jnp.tanh(c_root)
    g = jnp.dot(feat.astype(jnp.float32), w1_ref[...],
                preferred_element_type=jnp.float32)
    od = oda_ref[...] + odb_ref[...]
    s_ref[...] = g * lax.rsqrt(jnp.maximum(od, 1.0))


def _treelstm(x2d, c2d, wiou_t, b2d, uf_t, ufb2d, uiou_t, w1, oda, odb):
    full = lambda shp: pl.BlockSpec(shp, lambda i: (0, 0))
    return pl.pallas_call(
        _treelstm_body,
        grid=(N // TB,),
        in_specs=[
            pl.BlockSpec((TB, (KC + 1) * XS), lambda i: (i, 0)),
            pl.BlockSpec((TB, (KC + 1) * HS), lambda i: (i, 0)),
            full((XS, 3 * HS)),
            full((1, 3 * HS)),
            full((HS, HS)),
            full((1, HS)),
            full((HS, 3 * HS)),
            full((HS, 1)),
            pl.BlockSpec((TB, 1), lambda i: (i, 0)),
            pl.BlockSpec((TB, 1), lambda i: (i, 0)),
        ],
        out_specs=pl.BlockSpec((TB, 1), lambda i: (i, 0)),
        out_shape=jax.ShapeDtypeStruct((N, 1), jnp.float32),
    )(x2d, c2d, wiou_t, b2d, uf_t, ufb2d, uiou_t, w1, oda, odb)


FB = NP // 8


def _final_body(p0_ref, p1_ref, ida_ref, idb_ref, bc_ref, out_ref):
    agg = p0_ref[...] + p1_ref[...]
    ind = jnp.maximum(ida_ref[...] + idb_ref[...], 1.0)
    out_ref[...] = jax.nn.sigmoid(agg * lax.rsqrt(ind) + bc_ref[0, 0])


def _final(p0, p1, ida, idb, bc):
    bs = pl.BlockSpec((FB, 1), lambda i: (i, 0))
    return pl.pallas_call(
        _final_body,
        grid=(NP // FB,),
        in_specs=[bs, bs, bs, bs, pl.BlockSpec((1, 1), lambda i: (0, 0))],
        out_specs=bs,
        out_shape=jax.ShapeDtypeStruct((NP, 1), jnp.float32),
    )(p0, p1, ida, idb, bc)


def kernel(ast_x, h, c, cfg_edge_index, W_iou, U_iou, b_iou, U_f_w, U_f_b,
           gcn_w, gcn_b, lin1_w, lin1_b, lin2_w, lin2_b):
    del h
    f32 = jnp.float32

    m32 = lin1_w.T @ lin2_w.T
    w1 = gcn_w @ m32
    bconst = (gcn_b @ m32 + lin1_b @ lin2_w.T + lin2_b).reshape(1, 1)

    pad = jnp.full((2, NE_PAD - E), DUMMY, cfg_edge_index.dtype)
    edges = jnp.concatenate([cfg_edge_index, pad], axis=1).astype(jnp.int32)
    src_flat = edges[0]
    dst_flat = edges[1]

    zeros_np = jnp.zeros((NP,), f32)
    ones_pay = jnp.ones((EPW,), f32)

    deg = _get_sc_degrees()(src_flat, dst_flat, zeros_np,
                            ones_pay).reshape(NCORE, 2, NP)

    x2d = ast_x.reshape(N, (KC + 1) * XS)
    c2d = c.reshape(N, (KC + 1) * HS)
    oda = deg[0, 0, :N].reshape(N, 1)
    odb = deg[1, 0, :N].reshape(N, 1)
    s50 = _treelstm(x2d, c2d,
                    W_iou.T, b_iou.reshape(1, 3 * HS),
                    U_f_w.T, U_f_b.reshape(1, HS),
                    U_iou.T, w1, oda, odb)

    s_pad = jnp.concatenate([s50[:, 0], jnp.zeros((NP - N,), f32)])

    agg = _get_sc_scatter()(src_flat, dst_flat, s_pad,
                            zeros_np).reshape(NCORE, NP)

    out = _final(agg[0].reshape(NP, 1), agg[1].reshape(NP, 1),
                 deg[0, 1].reshape(NP, 1), deg[1, 1].reshape(NP, 1),
                 bconst.astype(f32))
    return out[:N]


# speedup vs baseline: 1.2076x; 1.0001x over previous
import functools

import jax
import jax.numpy as jnp
from jax import lax
from jax.experimental import pallas as pl
from jax.experimental.pallas import tpu as pltpu
from jax.experimental.pallas import tpu_sc as plsc

N = 50000
KC = 4
XS = 128
HS = 128
E = 800000

NP = 50176
DUMMY = N
NE_PAD = 819200
EROWS = 6400
NCORE = 2
NSUB = 16
EPW = NE_PAD // (NCORE * NSUB)
NP_SLICE = NP // NSUB

@functools.lru_cache(maxsize=None)
def _get_sc_degrees():
    mesh = plsc.VectorSubcoreMesh(
        core_axis_name="c", subcore_axis_name="s", num_cores=NCORE,
        num_subcores=NSUB)
    return functools.partial(
        pl.kernel,
        out_type=jax.ShapeDtypeStruct((NCORE * 2 * NP,), jnp.float32),
        mesh=mesh,
        scratch_types=[
            pltpu.VMEM((EPW,), jnp.int32),
            pltpu.VMEM((EPW,), jnp.int32),
            pltpu.VMEM((EPW,), jnp.float32),
            pltpu.VMEM((NP_SLICE,), jnp.float32),
            pltpu.VMEM_SHARED((NP,), jnp.float32),
            pltpu.VMEM_SHARED((NP,), jnp.float32),
        ],
    )(_sc_degrees_body)


def _sc_degrees_body(src_hbm, dst_hbm, zeros_hbm, ones_hbm, deg_out,
                     idx_src, idx_dst, ones_v, bounce, acc_o, acc_i):
    c = lax.axis_index("c")
    s = lax.axis_index("s")
    off = s * NP_SLICE

    pltpu.sync_copy(zeros_hbm.at[pl.ds(off, NP_SLICE)], bounce)
    pltpu.sync_copy(bounce, acc_o.at[pl.ds(off, NP_SLICE)])
    pltpu.sync_copy(bounce, acc_i.at[pl.ds(off, NP_SLICE)])

    base = (c * NSUB + s) * EPW
    pltpu.sync_copy(src_hbm.at[pl.ds(base, EPW)], idx_src)
    pltpu.sync_copy(dst_hbm.at[pl.ds(base, EPW)], idx_dst)
    pltpu.sync_copy(ones_hbm, ones_v)
    plsc.subcore_barrier()

    pltpu.sync_copy(ones_v, acc_o.at[idx_src], add=True)
    pltpu.sync_copy(ones_v, acc_i.at[idx_dst], add=True)
    plsc.subcore_barrier()
    pltpu.sync_copy(acc_o.at[pl.ds(off, NP_SLICE)], bounce)
    pltpu.sync_copy(bounce, deg_out.at[pl.ds((c * 2) * NP + off, NP_SLICE)])
    pltpu.sync_copy(acc_i.at[pl.ds(off, NP_SLICE)], bounce)
    pltpu.sync_copy(bounce, deg_out.at[pl.ds((c * 2 + 1) * NP + off, NP_SLICE)])


@functools.lru_cache(maxsize=None)
def _get_sc_scatter():
    mesh = plsc.VectorSubcoreMesh(
        core_axis_name="c", subcore_axis_name="s", num_cores=NCORE,
        num_subcores=NSUB)
    return functools.partial(
        pl.kernel,
        out_type=jax.ShapeDtypeStruct((NCORE * NP,), jnp.float32),
        mesh=mesh,
        scratch_types=[
            pltpu.VMEM((EPW,), jnp.int32),
            pltpu.VMEM((EPW,), jnp.int32),
            pltpu.VMEM((EPW,), jnp.float32),
            pltpu.VMEM((NP_SLICE,), jnp.float32),
            pltpu.VMEM_SHARED((NP,), jnp.float32),
            pltpu.SemaphoreType.DMA,
        ],
    )(_sc_scatter_body)


def _sc_scatter_body(src_hbm, dst_hbm, s_hbm, zeros_hbm, agg_out,
                     idx_src, idx_dst, vals, bounce, acc, sem):
    c = lax.axis_index("c")
    s = lax.axis_index("s")
    off = s * NP_SLICE

    pltpu.sync_copy(zeros_hbm.at[pl.ds(off, NP_SLICE)], bounce)
    pltpu.sync_copy(bounce, acc.at[pl.ds(off, NP_SLICE)])

    base = (c * NSUB + s) * EPW
    pltpu.sync_copy(src_hbm.at[pl.ds(base, EPW)], idx_src)
    pltpu.sync_copy(dst_hbm.at[pl.ds(base, EPW)], idx_dst)
    plsc.subcore_barrier()

    pltpu.async_copy(s_hbm.at[idx_src], vals, sem).wait()
    pltpu.sync_copy(vals, acc.at[idx_dst], add=True)
    plsc.subcore_barrier()
    pltpu.sync_copy(acc.at[pl.ds(off, NP_SLICE)], bounce)
    pltpu.sync_copy(bounce, agg_out.at[pl.ds(c * NP + off, NP_SLICE)])


TB = 2000


def _sigmoid(x):
    return jnp.tanh(x * 0.5) * 0.5 + 0.5


def _treelstm_body(x_ref, c_ref, wiou_ref, b_ref, uf_ref, ufb_ref,
                   uiou_ref, w1_ref, oda_ref, odb_ref, s_ref):
    bf16 = jnp.bfloat16
    b = b_ref[...]
    wiou = wiou_ref[...].astype(bf16)
    uf = uf_ref[...].astype(bf16)
    h_tild = None
    c_red = None
    for k in range(KC):
        lo = (k + 1) * HS
        x = x_ref[:, lo:lo + HS].astype(bf16)
        cc = c_ref[:, lo:lo + HS]
        iou = jnp.dot(x, wiou, preferred_element_type=jnp.float32) + b
        i_ = _sigmoid(iou[:, :HS])
        o_ = _sigmoid(iou[:, HS:2 * HS])
        u_ = jnp.tanh(iou[:, 2 * HS:])
        c_leaf = i_ * u_ + cc
        h_leaf = o_ * jnp.tanh(c_leaf)
        f = _sigmoid(
            jnp.dot(h_leaf.astype(bf16), uf, preferred_element_type=jnp.float32)
            + ufb_ref[...])
        fc = f * c_leaf
        h_tild = h_leaf if h_tild is None else h_tild + h_leaf
        c_red = fc if c_red is None else c_red + fc

    iou_r = jnp.dot(h_tild.astype(bf16), uiou_ref[...].astype(bf16),
                    preferred_element_type=jnp.float32) + b
    ir = _sigmoid(iou_r[:, :HS])
    orr = _sigmoid(iou_r[:, HS:2 * HS])
    ur = jnp.tanh(iou_r[:, 2 * HS:])
    c_root = ir * ur + c_red
    feat = orr * jnp.tanh(c_root)
    g = jnp.dot(feat, w1_ref[...], preferred_element_type=jnp.float32)
    od = oda_ref[...] + odb_ref[...]
    s_ref[...] = g * lax.rsqrt(jnp.maximum(od, 1.0))


def _treelstm(x2d, c2d, wiou_t, b2d, uf_t, ufb2d, uiou_t, w1, oda, odb):
    full = lambda shp: pl.BlockSpec(shp, lambda i: (0, 0))
    return pl.pallas_call(
        _treelstm_body,
        grid=(N // TB,),
        in_specs=[
            pl.BlockSpec((TB, (KC + 1) * XS), lambda i: (i, 0)),
            pl.BlockSpec((TB, (KC + 1) * HS), lambda i: (i, 0)),
            full((XS, 3 * HS)),
            full((1, 3 * HS)),
            full((HS, HS)),
            full((1, HS)),
            full((HS, 3 * HS)),
            full((HS, 1)),
            pl.BlockSpec((TB, 1), lambda i: (i, 0)),
            pl.BlockSpec((TB, 1), lambda i: (i, 0)),
        ],
        out_specs=pl.BlockSpec((TB, 1), lambda i: (i, 0)),
        out_shape=jax.ShapeDtypeStruct((N, 1), jnp.float32),
    )(x2d, c2d, wiou_t, b2d, uf_t, ufb2d, uiou_t, w1, oda, odb)


FB = NP // 8


def _final_body(p0_ref, p1_ref, ida_ref, idb_ref, bc_ref, out_ref):
    agg = p0_ref[...] + p1_ref[...]
    ind = jnp.maximum(ida_ref[...] + idb_ref[...], 1.0)
    out_ref[...] = jax.nn.sigmoid(agg * lax.rsqrt(ind) + bc_ref[0, 0])


def _final(p0, p1, ida, idb, bc):
    bs = pl.BlockSpec((FB, 1), lambda i: (i, 0))
    return pl.pallas_call(
        _final_body,
        grid=(NP // FB,),
        in_specs=[bs, bs, bs, bs, pl.BlockSpec((1, 1), lambda i: (0, 0))],
        out_specs=bs,
        out_shape=jax.ShapeDtypeStruct((NP, 1), jnp.float32),
    )(p0, p1, ida, idb, bc)


def kernel(ast_x, h, c, cfg_edge_index, W_iou, U_iou, b_iou, U_f_w, U_f_b,
           gcn_w, gcn_b, lin1_w, lin1_b, lin2_w, lin2_b):
    del h
    f32 = jnp.float32

    m32 = lin1_w.T @ lin2_w.T
    w1 = gcn_w @ m32
    bconst = (gcn_b @ m32 + lin1_b @ lin2_w.T + lin2_b).reshape(1, 1)

    pad = jnp.full((2, NE_PAD - E), DUMMY, cfg_edge_index.dtype)
    edges = jnp.concatenate([cfg_edge_index, pad], axis=1).astype(jnp.int32)
    src_flat = edges[0]
    dst_flat = edges[1]

    zeros_np = jnp.zeros((NP,), f32)
    ones_pay = jnp.ones((EPW,), f32)

    deg = _get_sc_degrees()(src_flat, dst_flat, zeros_np,
                            ones_pay).reshape(NCORE, 2, NP)

    x2d = ast_x.reshape(N, (KC + 1) * XS)
    c2d = c.reshape(N, (KC + 1) * HS)
    oda = deg[0, 0, :N].reshape(N, 1)
    odb = deg[1, 0, :N].reshape(N, 1)
    s50 = _treelstm(x2d, c2d,
                    W_iou.T, b_iou.reshape(1, 3 * HS),
                    U_f_w.T, U_f_b.reshape(1, HS),
                    U_iou.T, w1, oda, odb)

    s_pad = jnp.concatenate([s50[:, 0], jnp.zeros((NP - N,), f32)])

    agg = _get_sc_scatter()(src_flat, dst_flat, s_pad,
                            zeros_np).reshape(NCORE, NP)

    out = _final(agg[0].reshape(NP, 1), agg[1].reshape(NP, 1),
                 deg[0, 1].reshape(NP, 1), deg[1, 1].reshape(NP, 1),
                 bconst.astype(f32))
    return out[:N]
